# baseline (speedup 1.0000x reference)
import functools

import jax
import jax.numpy as jnp
from jax import lax
from jax.experimental import pallas as pl
from jax.experimental.pallas import tpu as pltpu
from jax.experimental.pallas import tpu_sc as plsc

L, B, C = 200, 1024, 8
VOCAB, D_SUB, D_EMBED = 100000, 32, 128
N_TOK = L * B
N_ROWS = N_TOK * C
PACK_W = D_SUB // 2

N_SEG = 5
L_SEG = L // N_SEG
SEG_TOK = N_TOK // N_SEG
SEG_ROWS = N_ROWS // N_SEG

NC, NS = 2, 16
NW = NC * NS
ROWS_PER_W = SEG_ROWS // NW
CHUNK = 1024
TOK_CHUNK = CHUNK // C
N_CHUNKS = ROWS_PER_W // CHUNK


def _gather_body(idx_hbm, table_hbm, out_hbm, idx_a, idx_b, rows_a, rows_b,
                 gsem_a, gsem_b, wsem_a, wsem_b):
    wid = lax.axis_index("s") * NC + lax.axis_index("c")
    base_w = wid * ROWS_PER_W
    n_half = N_CHUNKS // 2

    def load_and_fire(chunk, idx_v, rows_v, gsem):
        pltpu.sync_copy(idx_hbm.at[pl.ds(base_w + chunk * CHUNK, CHUNK)], idx_v)
        pltpu.async_copy(table_hbm.at[idx_v], rows_v, gsem)

    def write_copies(rows_v, out_base, wsem):
        return [
            pltpu.make_async_copy(
                rows_v.at[pl.ds(c * TOK_CHUNK, TOK_CHUNK), :],
                out_hbm.at[pl.ds(out_base, TOK_CHUNK), pl.ds(c * PACK_W, PACK_W)],
                wsem,
            )
            for c in range(C)
        ]

    def fire_writes(rows_v, chunk, wsem):
        for cp in write_copies(rows_v, (base_w + chunk * CHUNK) // C, wsem):
            cp.start()

    def drain_writes(rows_v, chunk, wsem):
        for cp in write_copies(rows_v, (base_w + chunk * CHUNK) // C, wsem):
            cp.wait()

    load_and_fire(0, idx_a, rows_a, gsem_a)

    def step(i, carry):
        pltpu.make_async_copy(table_hbm.at[idx_a], rows_a, gsem_a).wait()
        fire_writes(rows_a, 2 * i, wsem_a)

        @pl.when(i > 0)
        def _():
            drain_writes(rows_b, 2 * i - 1, wsem_b)

        load_and_fire(2 * i + 1, idx_b, rows_b, gsem_b)
        pltpu.make_async_copy(table_hbm.at[idx_b], rows_b, gsem_b).wait()
        fire_writes(rows_b, 2 * i + 1, wsem_b)

        @pl.when(i < n_half - 1)
        def _():
            drain_writes(rows_a, 2 * i, wsem_a)
            load_and_fire(2 * i + 2, idx_a, rows_a, gsem_a)

        return carry

    lax.fori_loop(0, n_half, step, 0)
    drain_writes(rows_a, N_CHUNKS - 2, wsem_a)
    drain_writes(rows_b, N_CHUNKS - 1, wsem_b)


_sc_gather = functools.partial(
    pl.kernel,
    out_type=jax.ShapeDtypeStruct((SEG_TOK, 128), jnp.int32),
    mesh=plsc.VectorSubcoreMesh(core_axis_name="c", subcore_axis_name="s"),
    scratch_types=[
        pltpu.VMEM((CHUNK,), jnp.int32),
        pltpu.VMEM((CHUNK,), jnp.int32),
        pltpu.VMEM((CHUNK, PACK_W), jnp.int32),
        pltpu.VMEM((CHUNK, PACK_W), jnp.int32),
        pltpu.SemaphoreType.DMA,
        pltpu.SemaphoreType.DMA,
        pltpu.SemaphoreType.DMA,
        pltpu.SemaphoreType.DMA,
    ],
    compiler_params=pltpu.CompilerParams(use_tc_tiling_on_sc=False),
)(_gather_body)


MM_TOK = 1024
MM_GRP = 5
SEG_BLOCKS = SEG_TOK // (MM_TOK * MM_GRP)


def _mm_compute(x_ref, wlo_ref, whi_ref, b_ref, o_ref):
    for g in range(MM_GRP):
        xi = x_ref[g * MM_TOK : (g + 1) * MM_TOK, :]
        xlo = lax.bitcast_convert_type(xi << 16, jnp.float32)
        xhi = lax.bitcast_convert_type(
            xi & jnp.int32(-65536), jnp.float32
        )
        o_ref[g * MM_TOK : (g + 1) * MM_TOK, :] = (
            jnp.dot(xlo, wlo_ref[...], preferred_element_type=jnp.float32)
            + jnp.dot(xhi, whi_ref[...], preferred_element_type=jnp.float32)
            + b_ref[0, :]
        )


def _mm_body_first(x_ref, wlo_ref, whi_ref, b_ref, o_ref):
    _mm_compute(x_ref, wlo_ref, whi_ref, b_ref, o_ref)


def _mm_body_chained(x_ref, wlo_ref, whi_ref, b_ref, acc_ref, o_ref):
    del acc_ref
    _mm_compute(x_ref, wlo_ref, whi_ref, b_ref, o_ref)


def _tc_matmul_seg(seg, x, wlo, whi, b, acc=None):
    blk = MM_TOK * MM_GRP

    def out_map(i, s=seg):
        return (s * SEG_BLOCKS + i, 0)

    in_specs = [
        pl.BlockSpec((blk, 128), lambda i: (i, 0)),
        pl.BlockSpec((128, D_EMBED), lambda i: (0, 0)),
        pl.BlockSpec((128, D_EMBED), lambda i: (0, 0)),
        pl.BlockSpec((1, D_EMBED), lambda i: (0, 0)),
    ]
    args = (x, wlo, whi, b)
    if acc is None:
        body = _mm_body_first
        aliases = {}
    else:
        body = _mm_body_chained
        in_specs = in_specs + [pl.BlockSpec(memory_space=pl.ANY)]
        args = args + (acc,)
        aliases = {4: 0}
    return pl.pallas_call(
        body,
        grid=(SEG_BLOCKS,),
        in_specs=in_specs,
        out_specs=pl.BlockSpec((blk, D_EMBED), out_map),
        out_shape=jax.ShapeDtypeStruct((N_TOK, D_EMBED), jnp.float32),
        input_output_aliases=aliases,
    )(*args)


def _pack_table(table):
    u = lax.bitcast_convert_type(table, jnp.int32).reshape(VOCAB // 4, 4, 2, PACK_W)
    r = (u + jnp.int32(0x7FFF) + ((u >> 16) & 1)) >> 16
    lo = r[:, :, 0, :] & jnp.int32(0xFFFF)
    hi = r[:, :, 1, :] << 16
    return (lo | hi).reshape(VOCAB // 8, 128).reshape(VOCAB, PACK_W)


@jax.jit
def kernel(input_ids, table, W_trans, b_trans):
    table_packed = _pack_table(table)
    wt = W_trans.T
    wlo = wt.reshape(C, 2, PACK_W, D_EMBED)[:, 0].reshape(128, D_EMBED)
    whi = wt.reshape(C, 2, PACK_W, D_EMBED)[:, 1].reshape(128, D_EMBED)
    bias = b_trans.reshape(1, D_EMBED)
    stagings = []
    for s in range(N_SEG):
        ids_s = input_ids.reshape(N_TOK, C)[s * SEG_TOK : (s + 1) * SEG_TOK]
        idx_flat = (
            ids_s.reshape(SEG_TOK // TOK_CHUNK, TOK_CHUNK, C)
            .transpose(0, 2, 1)
            .reshape(SEG_ROWS)
        )
        stagings.append(_sc_gather(idx_flat, table_packed))
    out = None
    for s in range(N_SEG):
        out = _tc_matmul_seg(s, stagings[s], wlo, whi, bias, acc=out)
    return out.reshape(L, B, D_EMBED)


# speedup vs baseline: 1.8161x; 1.8161x over previous
import functools

import jax
import jax.numpy as jnp
from jax import lax
from jax.experimental import pallas as pl
from jax.experimental.pallas import tpu as pltpu
from jax.experimental.pallas import tpu_sc as plsc

L, B, C = 200, 1024, 8
VOCAB, D_SUB, D_EMBED = 100000, 32, 128
N_TOK = L * B
N_ROWS = N_TOK * C
PACK_W = D_SUB // 2

N_SEG = 5
L_SEG = L // N_SEG
SEG_TOK = N_TOK // N_SEG
SEG_ROWS = N_ROWS // N_SEG

NC, NS = 2, 16
NW = NC * NS
ROWS_PER_W = SEG_ROWS // NW
CHUNK = 1024
TOK_CHUNK = CHUNK // C
N_CHUNKS = ROWS_PER_W // CHUNK


def _gather_body(idx_hbm, table_hbm, out_hbm, idx_a, idx_b, rows_a, rows_b,
                 gsem_a, gsem_b, wsem_a, wsem_b):
    wid = lax.axis_index("s") * NC + lax.axis_index("c")
    base_w = wid * ROWS_PER_W
    n_half = N_CHUNKS // 2

    def load_and_fire(chunk, idx_v, rows_v, gsem):
        pltpu.sync_copy(idx_hbm.at[pl.ds(base_w + chunk * CHUNK, CHUNK)], idx_v)
        pltpu.async_copy(table_hbm.at[idx_v], rows_v, gsem)

    def write_copies(rows_v, out_base, wsem):
        return [
            pltpu.make_async_copy(
                rows_v.at[pl.ds(c * TOK_CHUNK, TOK_CHUNK), :],
                out_hbm.at[pl.ds(out_base, TOK_CHUNK), pl.ds(c * PACK_W, PACK_W)],
                wsem,
            )
            for c in range(C)
        ]

    def fire_writes(rows_v, chunk, wsem):
        for cp in write_copies(rows_v, (base_w + chunk * CHUNK) // C, wsem):
            cp.start()

    def drain_writes(rows_v, chunk, wsem):
        for cp in write_copies(rows_v, (base_w + chunk * CHUNK) // C, wsem):
            cp.wait()

    load_and_fire(0, idx_a, rows_a, gsem_a)

    def step(i, carry):
        pltpu.make_async_copy(table_hbm.at[idx_a], rows_a, gsem_a).wait()
        fire_writes(rows_a, 2 * i, wsem_a)

        @pl.when(i > 0)
        def _():
            drain_writes(rows_b, 2 * i - 1, wsem_b)

        load_and_fire(2 * i + 1, idx_b, rows_b, gsem_b)
        pltpu.make_async_copy(table_hbm.at[idx_b], rows_b, gsem_b).wait()
        fire_writes(rows_b, 2 * i + 1, wsem_b)

        @pl.when(i < n_half - 1)
        def _():
            drain_writes(rows_a, 2 * i, wsem_a)
            load_and_fire(2 * i + 2, idx_a, rows_a, gsem_a)

        return carry

    lax.fori_loop(0, n_half, step, 0)
    drain_writes(rows_a, N_CHUNKS - 2, wsem_a)
    drain_writes(rows_b, N_CHUNKS - 1, wsem_b)


_sc_gather = functools.partial(
    pl.kernel,
    out_type=jax.ShapeDtypeStruct((SEG_TOK, 128), jnp.int32),
    mesh=plsc.VectorSubcoreMesh(core_axis_name="c", subcore_axis_name="s"),
    scratch_types=[
        pltpu.VMEM((CHUNK,), jnp.int32),
        pltpu.VMEM((CHUNK,), jnp.int32),
        pltpu.VMEM((CHUNK, PACK_W), jnp.int32),
        pltpu.VMEM((CHUNK, PACK_W), jnp.int32),
        pltpu.SemaphoreType.DMA,
        pltpu.SemaphoreType.DMA,
        pltpu.SemaphoreType.DMA,
        pltpu.SemaphoreType.DMA,
    ],
    compiler_params=pltpu.CompilerParams(use_tc_tiling_on_sc=False),
)(_gather_body)


MM_TOK = 1024
MM_GRP = 5
SEG_BLOCKS = SEG_TOK // (MM_TOK * MM_GRP)


def _mm_compute(x_ref, wlo_ref, whi_ref, b_ref, o_ref):
    for g in range(MM_GRP):
        xi = x_ref[g * MM_TOK : (g + 1) * MM_TOK, :]
        xlo = lax.bitcast_convert_type(xi << 16, jnp.float32)
        xhi = lax.bitcast_convert_type(
            xi & jnp.int32(-65536), jnp.float32
        )
        o_ref[g * MM_TOK : (g + 1) * MM_TOK, :] = (
            jnp.dot(xlo, wlo_ref[...], preferred_element_type=jnp.float32)
            + jnp.dot(xhi, whi_ref[...], preferred_element_type=jnp.float32)
            + b_ref[0, :]
        )


def _mm_body_first(x_ref, wlo_ref, whi_ref, b_ref, o_ref):
    _mm_compute(x_ref, wlo_ref, whi_ref, b_ref, o_ref)


def _mm_body_chained(x_ref, wlo_ref, whi_ref, b_ref, acc_ref, o_ref):
    del acc_ref
    _mm_compute(x_ref, wlo_ref, whi_ref, b_ref, o_ref)


def _tc_matmul_seg(seg, x, wlo, whi, b, acc=None):
    blk = MM_TOK * MM_GRP

    def out_map(i, s=seg):
        return (s * SEG_BLOCKS + i, 0)

    in_specs = [
        pl.BlockSpec((blk, 128), lambda i: (i, 0)),
        pl.BlockSpec((128, D_EMBED), lambda i: (0, 0)),
        pl.BlockSpec((128, D_EMBED), lambda i: (0, 0)),
        pl.BlockSpec((1, D_EMBED), lambda i: (0, 0)),
    ]
    args = (x, wlo, whi, b)
    if acc is None:
        body = _mm_body_first
        aliases = {}
    else:
        body = _mm_body_chained
        in_specs = in_specs + [pl.BlockSpec(memory_space=pl.ANY)]
        args = args + (acc,)
        aliases = {4: 0}
    return pl.pallas_call(
        body,
        grid=(SEG_BLOCKS,),
        in_specs=in_specs,
        out_specs=pl.BlockSpec((blk, D_EMBED), out_map),
        out_shape=jax.ShapeDtypeStruct((N_TOK, D_EMBED), jnp.float32),
        input_output_aliases=aliases,
    )(*args)


PACK_BLK = 5000


def _pack_body(x_ref, o_ref):
    u = lax.bitcast_convert_type(x_ref[...], jnp.int32)
    r = (u + jnp.int32(0x7FFF) + ((u >> 16) & 1)) >> 16
    o_ref[...] = (r[:, :PACK_W] & jnp.int32(0xFFFF)) | (r[:, PACK_W:] << 16)


def _pack_table(table):
    return pl.pallas_call(
        _pack_body,
        grid=(VOCAB // PACK_BLK,),
        in_specs=[pl.BlockSpec((PACK_BLK, D_SUB), lambda i: (i, 0))],
        out_specs=pl.BlockSpec((PACK_BLK, PACK_W), lambda i: (i, 0)),
        out_shape=jax.ShapeDtypeStruct((VOCAB, PACK_W), jnp.int32),
    )(table)


@jax.jit
def kernel(input_ids, table, W_trans, b_trans):
    table_packed = _pack_table(table)
    wt = W_trans.T
    wlo = wt.reshape(C, 2, PACK_W, D_EMBED)[:, 0].reshape(128, D_EMBED)
    whi = wt.reshape(C, 2, PACK_W, D_EMBED)[:, 1].reshape(128, D_EMBED)
    bias = b_trans.reshape(1, D_EMBED)
    stagings = []
    for s in range(N_SEG):
        ids_s = input_ids.reshape(N_TOK, C)[s * SEG_TOK : (s + 1) * SEG_TOK]
        idx_flat = (
            ids_s.reshape(SEG_TOK // TOK_CHUNK, TOK_CHUNK, C)
            .transpose(0, 2, 1)
            .reshape(SEG_ROWS)
        )
        stagings.append(_sc_gather(idx_flat, table_packed))
    out = None
    for s in range(N_SEG):
        out = _tc_matmul_seg(s, stagings[s], wlo, whi, bias, acc=out)
    return out.reshape(L, B, D_EMBED)


# speedup vs baseline: 1.8514x; 1.0194x over previous
import functools

import jax
import jax.numpy as jnp
from jax import lax
from jax.experimental import pallas as pl
from jax.experimental.pallas import tpu as pltpu
from jax.experimental.pallas import tpu_sc as plsc

L, B, C = 200, 1024, 8
VOCAB, D_SUB, D_EMBED = 100000, 32, 128
N_TOK = L * B
N_ROWS = N_TOK * C

N_SEG = 5
L_SEG = L // N_SEG
SEG_ROWS = N_ROWS // N_SEG
SEG_OUT = SEG_ROWS // 4

NC, NS = 2, 16
NW = NC * NS
ROWS_PER_W = SEG_ROWS // NW
CHUNK = 1024
OUT_CHUNK = CHUNK // 4
N_CHUNKS = ROWS_PER_W // CHUNK


def _gather_body(idx_hbm, table_hbm, out_hbm, idx_a, idx_b, rows_a, rows_b,
                 gsem_a, gsem_b, wsem_a, wsem_b):
    wid = lax.axis_index("s") * NC + lax.axis_index("c")
    base_w = wid * ROWS_PER_W
    n_half = N_CHUNKS // 2

    def load_and_fire(chunk, idx_v, rows_v, gsem):
        pltpu.sync_copy(idx_hbm.at[pl.ds(base_w + chunk * CHUNK, CHUNK)], idx_v)
        pltpu.async_copy(table_hbm.at[idx_v], rows_v, gsem)

    def write_copies(rows_v, out_base, wsem):
        return [
            pltpu.make_async_copy(
                rows_v.at[pl.ds(j * OUT_CHUNK, OUT_CHUNK), :],
                out_hbm.at[pl.ds(out_base, OUT_CHUNK), pl.ds(j * D_SUB, D_SUB)],
                wsem,
            )
            for j in range(4)
        ]

    def fire_writes(rows_v, chunk, wsem):
        for cp in write_copies(rows_v, (base_w + chunk * CHUNK) // 4, wsem):
            cp.start()

    def drain_writes(rows_v, chunk, wsem):
        for cp in write_copies(rows_v, (base_w + chunk * CHUNK) // 4, wsem):
            cp.wait()

    load_and_fire(0, idx_a, rows_a, gsem_a)

    def step(i, carry):
        pltpu.make_async_copy(table_hbm.at[idx_a], rows_a, gsem_a).wait()
        fire_writes(rows_a, 2 * i, wsem_a)

        @pl.when(i > 0)
        def _():
            drain_writes(rows_b, 2 * i - 1, wsem_b)

        load_and_fire(2 * i + 1, idx_b, rows_b, gsem_b)
        pltpu.make_async_copy(table_hbm.at[idx_b], rows_b, gsem_b).wait()
        fire_writes(rows_b, 2 * i + 1, wsem_b)

        @pl.when(i < n_half - 1)
        def _():
            drain_writes(rows_a, 2 * i, wsem_a)
            load_and_fire(2 * i + 2, idx_a, rows_a, gsem_a)

        return carry

    lax.fori_loop(0, n_half, step, 0)
    drain_writes(rows_a, N_CHUNKS - 2, wsem_a)
    drain_writes(rows_b, N_CHUNKS - 1, wsem_b)


_sc_gather = functools.partial(
    pl.kernel,
    out_type=jax.ShapeDtypeStruct((SEG_OUT, 128), jnp.float32),
    mesh=plsc.VectorSubcoreMesh(core_axis_name="c", subcore_axis_name="s"),
    scratch_types=[
        pltpu.VMEM((CHUNK,), jnp.int32),
        pltpu.VMEM((CHUNK,), jnp.int32),
        pltpu.VMEM((CHUNK, D_SUB), jnp.float32),
        pltpu.VMEM((CHUNK, D_SUB), jnp.float32),
        pltpu.SemaphoreType.DMA,
        pltpu.SemaphoreType.DMA,
        pltpu.SemaphoreType.DMA,
        pltpu.SemaphoreType.DMA,
    ],
    compiler_params=pltpu.CompilerParams(use_tc_tiling_on_sc=False),
)(_gather_body)


MM_TOK = 1024
MM_GRP = 5
SEG_BLOCKS = L_SEG // MM_GRP


def _mm_compute(x_ref, w_ref, b_ref, o_ref):
    for g in range(MM_GRP):
        x0 = x_ref[2 * g * MM_TOK : (2 * g + 1) * MM_TOK, :]
        x1 = x_ref[(2 * g + 1) * MM_TOK : (2 * g + 2) * MM_TOK, :]
        x = jnp.concatenate([x0, x1], axis=1)
        o_ref[g * MM_TOK : (g + 1) * MM_TOK, :] = (
            jnp.dot(x, w_ref[...], preferred_element_type=jnp.float32)
            + b_ref[0, :]
        )


def _mm_body_first(x_ref, w_ref, b_ref, o_ref):
    _mm_compute(x_ref, w_ref, b_ref, o_ref)


def _mm_body_chained(x_ref, w_ref, b_ref, acc_ref, o_ref):
    del acc_ref
    _mm_compute(x_ref, w_ref, b_ref, o_ref)


def _tc_matmul_seg(seg, x, w, b, acc=None):
    blk = MM_TOK * MM_GRP

    def out_map(i, s=seg):
        return (s * SEG_BLOCKS + i, 0)

    in_specs = [
        pl.BlockSpec((2 * blk, 128), lambda i: (i, 0)),
        pl.BlockSpec((256, D_EMBED), lambda i: (0, 0)),
        pl.BlockSpec((1, D_EMBED), lambda i: (0, 0)),
    ]
    args = (x, w, b)
    if acc is None:
        body = _mm_body_first
        aliases = {}
    else:
        body = _mm_body_chained
        in_specs = in_specs + [pl.BlockSpec(memory_space=pl.ANY)]
        args = args + (acc,)
        aliases = {3: 0}
    return pl.pallas_call(
        body,
        grid=(SEG_BLOCKS,),
        in_specs=in_specs,
        out_specs=pl.BlockSpec((blk, D_EMBED), out_map),
        out_shape=jax.ShapeDtypeStruct((N_TOK, D_EMBED), jnp.float32),
        input_output_aliases=aliases,
    )(*args)


@jax.jit
def kernel(input_ids, table, W_trans, b_trans):
    wt = W_trans.T
    bias = b_trans.reshape(1, D_EMBED)
    stagings = []
    for s in range(N_SEG):
        ids_s = input_ids[s * L_SEG : (s + 1) * L_SEG]
        idx_flat = (
            ids_s.reshape(L_SEG, 4, 256, 2, 4)
            .transpose(0, 3, 1, 4, 2)
            .reshape(SEG_ROWS)
        )
        stagings.append(_sc_gather(idx_flat, table))
    out = None
    for s in range(N_SEG):
        out = _tc_matmul_seg(s, stagings[s], wt, bias, acc=out)
    return out.reshape(L, B, D_EMBED)
